# baseline (speedup 1.0000x reference)
import functools

import jax
import jax.numpy as jnp
from jax import lax
from jax.experimental import pallas as pl
from jax.experimental.pallas import tpu as pltpu
from jax.experimental.pallas import tpu_sc as plsc

_VOCAB = 1000000
_EMBED = 32
_BATCH = 4096
_NT = 16
_CHUNK = _BATCH // _NT
_NG = _CHUNK // 16
_NPART = 5

_mesh = plsc.VectorSubcoreMesh(
    core_axis_name="c", subcore_axis_name="s", num_cores=1)


def _log_body(co_ref, out_ref):
  out_ref[...] = jnp.log(co_ref[...])


def _log_tc(co):
  return pl.pallas_call(
      _log_body,
      out_shape=jax.ShapeDtypeStruct((_BATCH // 128, 128), jnp.float32),
  )(co.reshape(_BATCH // 128, 128)).reshape(-1)


@functools.partial(
    pl.kernel,
    out_type=jax.ShapeDtypeStruct((16,), jnp.float32),
    mesh=_mesh,
    compiler_params=pltpu.CompilerParams(use_tc_tiling_on_sc=False),
    scratch_types=[
        pltpu.VMEM((_CHUNK,), jnp.int32),
        pltpu.VMEM((_CHUNK,), jnp.int32),
        pltpu.VMEM((_CHUNK, _EMBED), jnp.float32),
        pltpu.VMEM((_CHUNK, _EMBED), jnp.float32),
        pltpu.VMEM((_CHUNK,), jnp.float32),
        pltpu.VMEM((_CHUNK,), jnp.float32),
        pltpu.VMEM((_CHUNK,), jnp.float32),
        pltpu.VMEM((_CHUNK,), jnp.float32),
        pltpu.VMEM((_NPART * 16,), jnp.float32),
        pltpu.VMEM((_NT * _NPART * 16,), jnp.float32),
        pltpu.VMEM((16,), jnp.float32),
        pltpu.VMEM_SHARED((_NT * _NPART * 16,), jnp.float32),
        pltpu.SemaphoreType.DMA,
    ],
)
def _glove_sc(i_hbm, j_hbm, co_hbm, w_hbm, wv_hbm, ww_hbm, bv_hbm, bw_hbm,
              out_hbm, idx_i, idx_j, vi, wj, bvg, bwg, cov, wv_,
              pstage, pall, ostage, shared, sem):
  tid = lax.axis_index("s")
  base = tid * _CHUNK

  pltpu.sync_copy(i_hbm.at[pl.ds(base, _CHUNK)], idx_i)
  pltpu.sync_copy(j_hbm.at[pl.ds(base, _CHUNK)], idx_j)
  cp1 = pltpu.async_copy(wv_hbm.at[idx_i], vi, sem)
  cp2 = pltpu.async_copy(ww_hbm.at[idx_j], wj, sem)
  cp3 = pltpu.async_copy(bv_hbm.at[idx_i], bvg, sem)
  cp4 = pltpu.async_copy(bw_hbm.at[idx_j], bwg, sem)
  pltpu.sync_copy(co_hbm.at[pl.ds(base, _CHUNK)], cov)
  pltpu.sync_copy(w_hbm.at[pl.ds(base, _CHUNK)], wv_)
  cp1.wait()
  cp2.wait()
  cp3.wait()
  cp4.wait()

  lanes = lax.iota(jnp.int32, 16)
  zero = jnp.zeros((16,), jnp.float32)

  gdn = lax.GatherDimensionNumbers(
      offset_dims=(), collapsed_slice_dims=(0,), start_index_map=(0,))

  def _shuf(v, s):
    return lax.gather(v, (lanes ^ s)[:, None], dimension_numbers=gdn,
                      slice_sizes=(1,),
                      mode=lax.GatherScatterMode.PROMISE_IN_BOUNDS)

  def _rowsums(qs):
    s = 1
    while len(qs) > 1:
      nxt = []
      for m in range(0, len(qs), 2):
        a, b = qs[m], qs[m + 1]
        low = (lanes & s) == 0
        d = jnp.where(low, a, b)
        e = jnp.where(low, b, a)
        nxt.append(d + _shuf(e, s))
      qs = nxt
      s *= 2
    return qs[0]

  def _allsum(v):
    for s in (1, 2, 4, 8):
      v = v + _shuf(v, s)
    return v

  def grp_body(g, carry):
    s1, s2, s3, c1, c2 = carry
    o = g * 16
    qs = []
    for rl in range(16):
      r = o + rl
      qs.append(vi[r, pl.ds(0, 16)] * wj[r, pl.ds(0, 16)] +
                vi[r, pl.ds(16, 16)] * wj[r, pl.ds(16, 16)])
    svec = _rowsums(qs)
    sl = svec - cov[pl.ds(o, 16)]
    w = wv_[pl.ds(o, 16)]
    cv = bvg[pl.ds(o, 16)] + bwg[pl.ds(o, 16)]
    return (s1 + w * sl * sl, s2 + w * sl, s3 + w,
            c1 + cv, c2 + cv * cv)

  s1, s2, s3, c1, c2 = lax.fori_loop(
      0, _NG, grp_body, (zero, zero, zero, zero, zero))

  pstage[pl.ds(0, 16)] = s1
  pstage[pl.ds(16, 16)] = s2
  pstage[pl.ds(32, 16)] = s3
  pstage[pl.ds(48, 16)] = c1
  pstage[pl.ds(64, 16)] = c2
  pltpu.sync_copy(pstage, shared.at[pl.ds(tid * (_NPART * 16), _NPART * 16)])
  plsc.subcore_barrier()

  @pl.when(tid == 0)
  def _():
    pltpu.sync_copy(shared, pall)
    acc = [jnp.zeros((16,), jnp.float32) for _ in range(_NPART)]
    for t in range(_NT):
      for k in range(_NPART):
        acc[k] = acc[k] + pall[pl.ds(t * (_NPART * 16) + k * 16, 16)]
    ps1 = _allsum(acc[0])
    ps2 = _allsum(acc[1])
    ps3 = _allsum(acc[2])
    pc1 = _allsum(acc[3])
    pc2 = _allsum(acc[4])
    ostage[...] = (0.5 * _BATCH) * ps1 + pc1 * ps2 + 0.5 * pc2 * ps3
    pltpu.sync_copy(ostage, out_hbm)


def kernel(i, j, co_occur, weight, Wv, Ww, bv, bw):
  out = _glove_sc(i, j, _log_tc(co_occur), weight, Wv, Ww,
                  bv.reshape(-1), bw.reshape(-1))
  return out[0]


# speedup vs baseline: 10.9799x; 10.9799x over previous
import functools

import jax
import jax.numpy as jnp
from jax import lax
from jax.experimental import pallas as pl
from jax.experimental.pallas import tpu as pltpu
from jax.experimental.pallas import tpu_sc as plsc

_VOCAB = 1000000
_EMBED = 32
_BATCH = 4096
_NC = 2
_NT = 16
_CHUNK = _BATCH // (_NC * _NT)
_NBUF = 8

_mesh = plsc.VectorSubcoreMesh(
    core_axis_name="c", subcore_axis_name="s", num_cores=_NC)


def _log_body(co_ref, out_ref):
  out_ref[...] = jnp.log(co_ref[...])


def _log_tc(co):
  return pl.pallas_call(
      _log_body,
      out_shape=jax.ShapeDtypeStruct((_BATCH // 128, 128), jnp.float32),
  )(co.reshape(_BATCH // 128, 128)).reshape(-1)


def _combine_body(p_ref, out_ref):
  t = [p_ref[0, 0, k] + p_ref[1, 0, k] for k in range(5)]
  total = (0.5 * _BATCH) * t[0] + t[1] * t[3] + 0.5 * t[2] * t[4]
  out_ref[...] = jnp.full((8, 128), total, jnp.float32)


def _combine_tc(parts):
  return pl.pallas_call(
      _combine_body,
      out_shape=jax.ShapeDtypeStruct((8, 128), jnp.float32),
  )(parts)


@functools.partial(
    pl.kernel,
    out_type=jax.ShapeDtypeStruct((_NC, 8, 128), jnp.float32),
    mesh=_mesh,
    compiler_params=pltpu.CompilerParams(use_tc_tiling_on_sc=True),
    scratch_types=[
        pltpu.VMEM((_BATCH + 256,), jnp.int32),
        pltpu.VMEM((_BATCH + 256,), jnp.int32),
        pltpu.VMEM((_BATCH + 256,), jnp.float32),
        pltpu.VMEM((_BATCH + 256,), jnp.float32),
        pltpu.VMEM((_NBUF, _EMBED + 1, 128), jnp.float32),
        pltpu.VMEM((_NBUF, _EMBED + 1, 128), jnp.float32),
        pltpu.VMEM((_NBUF, 2, 128), jnp.float32),
        pltpu.VMEM((_NBUF, 2, 128), jnp.float32),
        pltpu.VMEM((8, 128), jnp.float32),
        pltpu.VMEM((_NT, 8, 128), jnp.float32),
        pltpu.VMEM_SHARED((_NT, 8, 128), jnp.float32),
        pltpu.SemaphoreType.DMA,
    ],
)
def _glove_sc(i_hbm, j_hbm, lco_hbm, w_hbm, wvT, wwT, bvT, bwT, out_hbm,
              idxi, idxj, lco, wgt, rvi, rvj, rbv, rbw, stage, redbuf,
              shared, sem):
  cid = lax.axis_index("c")
  sid = lax.axis_index("s")
  base = (sid * _NC + cid) * _CHUNK

  pltpu.sync_copy(i_hbm, idxi.at[pl.ds(0, _BATCH)])
  pltpu.sync_copy(j_hbm, idxj.at[pl.ds(0, _BATCH)])
  pltpu.sync_copy(lco_hbm, lco.at[pl.ds(0, _BATCH)])
  pltpu.sync_copy(w_hbm, wgt.at[pl.ds(0, _BATCH)])

  lanes = lax.iota(jnp.int32, 16)

  def scal(ref, n):
    return ref[pl.ds(n, 16)][0]

  def issue(n, b):
    vi = scal(idxi, base + n)
    vj = scal(idxj, base + n)
    ti = pl.multiple_of((vi >> 7) * 128, 128)
    tj = pl.multiple_of((vj >> 7) * 128, 128)
    pltpu.async_copy(wvT.at[:, pl.ds(ti, 128)], rvi.at[b, pl.ds(0, _EMBED)], sem)
    pltpu.async_copy(wwT.at[:, pl.ds(tj, 128)], rvj.at[b, pl.ds(0, _EMBED)], sem)
    pltpu.async_copy(bvT.at[:, pl.ds(ti, 128)], rbv.at[b, pl.ds(0, 1)], sem)
    pltpu.async_copy(bwT.at[:, pl.ds(tj, 128)], rbw.at[b, pl.ds(0, 1)], sem)

  def drain(b):
    pltpu.make_async_copy(
        wvT.at[:, pl.ds(0, 128)], rvi.at[b, pl.ds(0, _EMBED)], sem).wait()
    pltpu.make_async_copy(
        wwT.at[:, pl.ds(0, 128)], rvj.at[b, pl.ds(0, _EMBED)], sem).wait()
    pltpu.make_async_copy(
        bvT.at[:, pl.ds(0, 128)], rbv.at[b, pl.ds(0, 1)], sem).wait()
    pltpu.make_async_copy(
        bwT.at[:, pl.ds(0, 128)], rbw.at[b, pl.ds(0, 1)], sem).wait()

  for b in range(_NBUF):
    issue(b, b)

  def loop_body(g, carry):
    s1, s2, s3, c1, c2 = carry
    n0 = g * _NBUF
    for b in range(_NBUF):
      n = n0 + b
      drain(b)
      vi = scal(idxi, base + n)
      vj = scal(idxj, base + n)
      li = vi & 127
      lj = vj & 127
      q = jnp.zeros((16,), jnp.float32)
      for r in range(_EMBED):
        q = q + rvi[b, r, pl.ds(li, 16)] * rvj[b, r, pl.ds(lj, 16)]
      s = q[0]
      bvs = rbv[b, 0, pl.ds(li, 16)][0]
      bws = rbw[b, 0, pl.ds(lj, 16)][0]
      w = scal(wgt, base + n)
      sl = s - scal(lco, base + n)
      t = w * sl
      s1 = s1 + t * sl
      s2 = s2 + t
      s3 = s3 + w
      c = bvs + bws
      c1 = c1 + c
      c2 = c2 + c * c
      n2 = n + _NBUF

      @pl.when(n2 < _CHUNK)
      def _():
        issue(n2, b)

    return (s1, s2, s3, c1, c2)

  z = jnp.float32(0.0)
  s1, s2, s3, c1, c2 = lax.fori_loop(
      0, _CHUNK // _NBUF, loop_body, (z, z, z, z, z))

  pvec = jnp.zeros((16,), jnp.float32)
  for k, val in enumerate((s1, s2, s3, c1, c2)):
    pvec = jnp.where(lanes == k, jnp.full((16,), val, jnp.float32), pvec)
  stage[0, pl.ds(0, 16)] = pvec
  pltpu.sync_copy(stage, shared.at[sid])
  plsc.subcore_barrier()

  @pl.when(sid == 0)
  def _():
    pltpu.sync_copy(shared, redbuf)
    acc = jnp.zeros((16,), jnp.float32)
    for t in range(_NT):
      acc = acc + redbuf[t, 0, pl.ds(0, 16)]
    stage[0, pl.ds(0, 16)] = acc
    pltpu.sync_copy(stage, out_hbm.at[cid])


def kernel(i, j, co_occur, weight, Wv, Ww, bv, bw):
  parts = _glove_sc(i, j, _log_tc(co_occur), weight,
                    Wv.T, Ww.T, bv.T, bw.T)
  return _combine_tc(parts)[0, 0]
